# baseline (speedup 1.0000x reference)
import functools

import jax
import jax.numpy as jnp
from jax import lax
from jax.experimental import pallas as pl
from jax.experimental.pallas import tpu as pltpu

_CONTRACT_0 = (((0,), (0,)), ((), ()))


def _ceil_to(x, m):
    return ((x + m - 1) // m) * m


def _loss_kernel(e_ref, m_ref, a_ref, out_ref, *, d_p):
    a = a_ref[0].astype(jnp.bfloat16)
    e_ext = e_ref[...]
    e = e_ext[:, 0:d_p]

    c = lax.dot_general(e_ext, a, _CONTRACT_0,
                        preferred_element_type=jnp.float32)
    sum_a = jnp.sum(c[d_p:d_p + 1, :])

    b = jnp.dot(c[0:d_p, :].astype(jnp.bfloat16), e,
                preferred_element_type=jnp.float32)

    mk = m_ref[0]
    g = lax.dot_general(e, e, _CONTRACT_0, preferred_element_type=jnp.float32)
    y1 = jnp.dot(g, mk, preferred_element_type=jnp.float32)
    y2 = jnp.dot(mk, g, preferred_element_type=jnp.float32)
    t3 = jnp.sum(y1 * y2)

    dot_bm = jnp.sum(b * mk)
    val = sum_a - 2.0 * dot_bm + t3
    out_ref[...] = val + jnp.zeros((1, 1, 128), jnp.float32)


def kernel(E, M, A):
    n, d = E.shape
    K = M.shape[0]

    E = E.astype(jnp.float32)
    norms = jnp.sqrt(jnp.sum(E * E, axis=1, keepdims=True))
    E_n = E / jnp.maximum(norms, 1e-12)

    n_p = _ceil_to(n, 128)
    d_p = _ceil_to(d, 128)
    cols = d_p + 8

    e_ext = jnp.zeros((n_p, cols), jnp.float32)
    e_ext = e_ext.at[:n, :d].set(E_n)
    e_ext = e_ext.at[:n, d_p].set(1.0)
    e_bf = e_ext.astype(jnp.bfloat16)

    M_p = M.astype(jnp.float32)
    A_p = A
    if d_p != d:
        M_p = jnp.pad(M_p, ((0, 0), (0, d_p - d), (0, d_p - d)))
    if n_p != n:
        A_p = jnp.pad(A_p, ((0, 0), (0, n_p - n), (0, n_p - n)))

    out = pl.pallas_call(
        functools.partial(_loss_kernel, d_p=d_p),
        out_shape=jax.ShapeDtypeStruct((K, 1, 128), jnp.float32),
        grid=(K,),
        in_specs=[
            pl.BlockSpec((n_p, cols), lambda k: (0, 0)),
            pl.BlockSpec((1, d_p, d_p), lambda k: (k, 0, 0)),
            pl.BlockSpec((1, n_p, n_p), lambda k: (k, 0, 0)),
        ],
        out_specs=pl.BlockSpec((1, 1, 128), lambda k: (k, 0, 0)),
        compiler_params=pltpu.CompilerParams(
            dimension_semantics=("parallel",),
            vmem_limit_bytes=48 * 2 ** 20,
        ),
    )(e_bf, M_p, A_p)

    return jnp.sum(out[:, 0, 0])


# speedup vs baseline: 1.1688x; 1.1688x over previous
import functools

import jax
import jax.numpy as jnp
from jax import lax
from jax.experimental import pallas as pl
from jax.experimental.pallas import tpu as pltpu

_CONTRACT_0 = (((0,), (0,)), ((), ()))


def _ceil_to(x, m):
    return ((x + m - 1) // m) * m


def _loss_kernel(e_ref, m_ref, a_ref, out_ref, *, d_p):
    n_p = e_ref.shape[0]
    e = e_ref[...]

    sq = e * e
    nrm2 = jnp.dot(sq, jnp.ones((d_p, 128), jnp.float32),
                   preferred_element_type=jnp.float32)
    inv = lax.rsqrt(jnp.maximum(nrm2, 1e-24))
    e_nbf = (e * inv).astype(jnp.bfloat16)
    e_ext = jnp.concatenate(
        [e_nbf, jnp.ones((n_p, 8), jnp.bfloat16)], axis=1)

    a = a_ref[0].astype(jnp.bfloat16)

    c = lax.dot_general(e_ext, a, _CONTRACT_0,
                        preferred_element_type=jnp.float32)
    sum_a = jnp.sum(c[d_p:d_p + 1, :])

    b = jnp.dot(c[0:d_p, :].astype(jnp.bfloat16), e_nbf,
                preferred_element_type=jnp.float32)

    mk = m_ref[0]
    g = lax.dot_general(e_ext, e_ext, _CONTRACT_0,
                        preferred_element_type=jnp.float32)[0:d_p, 0:d_p]
    y1 = jnp.dot(g, mk, preferred_element_type=jnp.float32)
    y2 = jnp.dot(mk, g, preferred_element_type=jnp.float32)
    t3 = jnp.sum(y1 * y2)

    dot_bm = jnp.sum(b * mk)
    val = sum_a - 2.0 * dot_bm + t3
    out_ref[...] = val + jnp.zeros((1, 1, 128), jnp.float32)


def kernel(E, M, A):
    n, d = E.shape
    K = M.shape[0]

    n_p = _ceil_to(n, 128)
    d_p = _ceil_to(d, 128)

    E_p = E.astype(jnp.float32)
    M_p = M.astype(jnp.float32)
    A_p = A
    if d_p != d:
        E_p = jnp.pad(E_p, ((0, 0), (0, d_p - d)))
        M_p = jnp.pad(M_p, ((0, 0), (0, d_p - d), (0, d_p - d)))
    if n_p != n:
        E_p = jnp.pad(E_p, ((0, n_p - n), (0, 0)))
        A_p = jnp.pad(A_p, ((0, 0), (0, n_p - n), (0, n_p - n)))

    out = pl.pallas_call(
        functools.partial(_loss_kernel, d_p=d_p),
        out_shape=jax.ShapeDtypeStruct((K, 1, 128), jnp.float32),
        grid=(K,),
        in_specs=[
            pl.BlockSpec((n_p, d_p), lambda k: (0, 0)),
            pl.BlockSpec((1, d_p, d_p), lambda k: (k, 0, 0)),
            pl.BlockSpec((1, n_p, n_p), lambda k: (k, 0, 0)),
        ],
        out_specs=pl.BlockSpec((1, 1, 128), lambda k: (k, 0, 0)),
        compiler_params=pltpu.CompilerParams(
            dimension_semantics=("parallel",),
            vmem_limit_bytes=48 * 2 ** 20,
        ),
    )(E_p, M_p, A_p)

    return jnp.sum(out[:, 0, 0])


# speedup vs baseline: 1.5714x; 1.3445x over previous
import functools

import jax
import jax.numpy as jnp
from jax import lax
from jax.experimental import pallas as pl
from jax.experimental.pallas import tpu as pltpu

_CONTRACT_0 = (((0,), (0,)), ((), ()))


def _ceil_to(x, m):
    return ((x + m - 1) // m) * m


def _loss_kernel(e_ref, m_ref, a_ref, out_ref, *, d_p, kb):
    n_p = e_ref.shape[0]
    e = e_ref[...]

    sq = e * e
    nrm2 = jnp.dot(sq, jnp.ones((d_p, 128), jnp.float32),
                   preferred_element_type=jnp.float32)
    inv = lax.rsqrt(jnp.maximum(nrm2, 1e-24))
    e_nbf = (e * inv).astype(jnp.bfloat16)
    e_ext = jnp.concatenate(
        [e_nbf, jnp.ones((n_p, 8), jnp.bfloat16)], axis=1)

    g = lax.dot_general(e_ext, e_ext, _CONTRACT_0,
                        preferred_element_type=jnp.float32)[0:d_p, 0:d_p]

    val = jnp.float32(0.0)
    for kk in range(kb):
        a = a_ref[kk].astype(jnp.bfloat16)
        c = lax.dot_general(e_ext, a, _CONTRACT_0,
                            preferred_element_type=jnp.float32)
        sum_a = jnp.sum(c[d_p:d_p + 1, :])
        b = jnp.dot(c[0:d_p, :].astype(jnp.bfloat16), e_nbf,
                    preferred_element_type=jnp.float32)
        mk = m_ref[kk]
        y1 = jnp.dot(g, mk, preferred_element_type=jnp.float32)
        y2 = jnp.dot(mk, g, preferred_element_type=jnp.float32)
        t3 = jnp.sum(y1 * y2)
        dot_bm = jnp.sum(b * mk)
        val = val + sum_a - 2.0 * dot_bm + t3

    out_ref[...] = val + jnp.zeros((1, 1, 128), jnp.float32)


def kernel(E, M, A):
    n, d = E.shape
    K = M.shape[0]

    n_p = _ceil_to(n, 128)
    d_p = _ceil_to(d, 128)
    kb = 2 if K % 2 == 0 else 1
    gk = K // kb

    E_p = E.astype(jnp.float32)
    M_p = M.astype(jnp.float32)
    A_p = A
    if d_p != d:
        E_p = jnp.pad(E_p, ((0, 0), (0, d_p - d)))
        M_p = jnp.pad(M_p, ((0, 0), (0, d_p - d), (0, d_p - d)))
    if n_p != n:
        E_p = jnp.pad(E_p, ((0, n_p - n), (0, 0)))
        A_p = jnp.pad(A_p, ((0, 0), (0, n_p - n), (0, n_p - n)))

    out = pl.pallas_call(
        functools.partial(_loss_kernel, d_p=d_p, kb=kb),
        out_shape=jax.ShapeDtypeStruct((gk, 1, 128), jnp.float32),
        grid=(gk,),
        in_specs=[
            pl.BlockSpec((n_p, d_p), lambda j: (0, 0)),
            pl.BlockSpec((kb, d_p, d_p), lambda j: (j, 0, 0)),
            pl.BlockSpec((kb, n_p, n_p), lambda j: (j, 0, 0)),
        ],
        out_specs=pl.BlockSpec((1, 1, 128), lambda j: (j, 0, 0)),
        compiler_params=pltpu.CompilerParams(
            dimension_semantics=("parallel",),
            vmem_limit_bytes=48 * 2 ** 20,
        ),
    )(E_p, M_p, A_p)

    return jnp.sum(out[:, 0, 0])


# speedup vs baseline: 1.7569x; 1.1181x over previous
import functools

import jax
import jax.numpy as jnp
from jax import lax
from jax.experimental import pallas as pl
from jax.experimental.pallas import tpu as pltpu

_CONTRACT_0 = (((0,), (0,)), ((), ()))


def _ceil_to(x, m):
    return ((x + m - 1) // m) * m


def _loss_kernel(e_ref, m_ref, a_ref, out_ref, *, d_p, kb):
    n_p = e_ref.shape[0]
    e = e_ref[...]

    sq = e * e
    nrm2 = jnp.dot(sq, jnp.ones((d_p, 128), jnp.float32),
                   preferred_element_type=jnp.float32)
    inv = lax.rsqrt(jnp.maximum(nrm2, 1e-24))
    e_nbf = (e * inv).astype(jnp.bfloat16)
    e_ext = jnp.concatenate(
        [e_nbf, jnp.ones((n_p, 8), jnp.bfloat16)], axis=1)

    g = lax.dot_general(e_ext, e_ext, _CONTRACT_0,
                        preferred_element_type=jnp.float32)[0:d_p, 0:d_p]

    val = jnp.float32(0.0)
    for kk in range(kb):
        a = a_ref[kk].astype(jnp.bfloat16)
        c = lax.dot_general(e_ext, a, _CONTRACT_0,
                            preferred_element_type=jnp.float32)
        sum_a = jnp.sum(c[d_p:d_p + 1, :])
        b = jnp.dot(c[0:d_p, :].astype(jnp.bfloat16), e_nbf,
                    preferred_element_type=jnp.float32)
        mk = m_ref[kk]
        y1 = jnp.dot(g, mk, preferred_element_type=jnp.float32)
        y2 = jnp.dot(mk, g, preferred_element_type=jnp.float32)
        t3 = jnp.sum(y1 * y2)
        dot_bm = jnp.sum(b * mk)
        val = val + sum_a - 2.0 * dot_bm + t3

    out_ref[...] = val + jnp.zeros((1, 1, 128), jnp.float32)


def kernel(E, M, A):
    n, d = E.shape
    K = M.shape[0]

    n_p = _ceil_to(n, 128)
    d_p = _ceil_to(d, 128)
    kb = 4 if K % 4 == 0 else (2 if K % 2 == 0 else 1)
    gk = K // kb

    E_p = E.astype(jnp.float32)
    M_p = M.astype(jnp.float32)
    A_p = A
    if d_p != d:
        E_p = jnp.pad(E_p, ((0, 0), (0, d_p - d)))
        M_p = jnp.pad(M_p, ((0, 0), (0, d_p - d), (0, d_p - d)))
    if n_p != n:
        E_p = jnp.pad(E_p, ((0, n_p - n), (0, 0)))
        A_p = jnp.pad(A_p, ((0, 0), (0, n_p - n), (0, n_p - n)))

    out = pl.pallas_call(
        functools.partial(_loss_kernel, d_p=d_p, kb=kb),
        out_shape=jax.ShapeDtypeStruct((gk, 1, 128), jnp.float32),
        grid=(gk,),
        in_specs=[
            pl.BlockSpec((n_p, d_p), lambda j: (0, 0)),
            pl.BlockSpec((kb, d_p, d_p), lambda j: (j, 0, 0)),
            pl.BlockSpec((kb, n_p, n_p), lambda j: (j, 0, 0)),
        ],
        out_specs=pl.BlockSpec((1, 1, 128), lambda j: (j, 0, 0)),
        compiler_params=pltpu.CompilerParams(
            dimension_semantics=("parallel",),
            vmem_limit_bytes=48 * 2 ** 20,
        ),
    )(E_p, M_p, A_p)

    return jnp.sum(out[:, 0, 0])


# speedup vs baseline: 1.7621x; 1.0029x over previous
import functools

import jax
import jax.numpy as jnp
from jax import lax
from jax.experimental import pallas as pl
from jax.experimental.pallas import tpu as pltpu

_CONTRACT_0 = (((0,), (0,)), ((), ()))


def _ceil_to(x, m):
    return ((x + m - 1) // m) * m


def _loss_kernel(e_ref, m_ref, a_ref, out_ref, *, d_p, kb):
    n_p = e_ref.shape[0]
    e = e_ref[...]

    sq = e * e
    nrm2 = jnp.dot(sq, jnp.ones((d_p, 128), jnp.float32),
                   preferred_element_type=jnp.float32)
    inv = lax.rsqrt(jnp.maximum(nrm2, 1e-24))
    e_nbf = (e * inv).astype(jnp.bfloat16)
    e_ext = jnp.concatenate(
        [e_nbf, jnp.ones((n_p, 8), jnp.bfloat16)], axis=1)

    g = lax.dot_general(e_ext, e_ext, _CONTRACT_0,
                        preferred_element_type=jnp.float32)[0:d_p, 0:d_p]

    val = jnp.float32(0.0)
    for kk in range(kb):
        a = a_ref[kk].astype(jnp.bfloat16)
        c = lax.dot_general(e_ext, a, _CONTRACT_0,
                            preferred_element_type=jnp.float32)
        sum_a = jnp.sum(c[d_p:d_p + 1, :])
        b = jnp.dot(c[0:d_p, :].astype(jnp.bfloat16), e_nbf,
                    preferred_element_type=jnp.float32)
        mk = m_ref[kk]
        y1 = jnp.dot(g, mk, preferred_element_type=jnp.float32)
        y2 = jnp.dot(mk, g, preferred_element_type=jnp.float32)
        t3 = jnp.sum(y1 * y2)
        dot_bm = jnp.sum(b * mk)
        val = val + sum_a - 2.0 * dot_bm + t3

    out_ref[...] = val + jnp.zeros((1, 1, 128), jnp.float32)


def kernel(E, M, A):
    n, d = E.shape
    K = M.shape[0]

    n_p = _ceil_to(n, 128)
    d_p = _ceil_to(d, 128)
    kb = 4 if K % 4 == 0 else (2 if K % 2 == 0 else 1)
    gk = K // kb

    E_p = E if E.dtype == jnp.float32 else E.astype(jnp.float32)
    M_p = M if M.dtype == jnp.float32 else M.astype(jnp.float32)
    A_p = A
    if d_p != d:
        E_p = jnp.pad(E_p, ((0, 0), (0, d_p - d)))
        M_p = jnp.pad(M_p, ((0, 0), (0, d_p - d), (0, d_p - d)))
    if n_p != n:
        E_p = jnp.pad(E_p, ((0, n_p - n), (0, 0)))
        A_p = jnp.pad(A_p, ((0, 0), (0, n_p - n), (0, n_p - n)))

    out = pl.pallas_call(
        functools.partial(_loss_kernel, d_p=d_p, kb=kb),
        out_shape=jax.ShapeDtypeStruct((gk, 1, 128), jnp.float32),
        grid=(gk,),
        in_specs=[
            pl.BlockSpec((n_p, d_p), lambda j: (0, 0)),
            pl.BlockSpec((kb, d_p, d_p), lambda j: (j, 0, 0)),
            pl.BlockSpec((kb, n_p, n_p), lambda j: (j, 0, 0)),
        ],
        out_specs=pl.BlockSpec((1, 1, 128), lambda j: (j, 0, 0)),
        compiler_params=pltpu.CompilerParams(
            dimension_semantics=("parallel",),
            vmem_limit_bytes=48 * 2 ** 20,
        ),
    )(E_p, M_p, A_p)

    return jnp.sum(out[:, 0, 0])
